# baseline (speedup 1.0000x reference)
import functools

import jax
import jax.numpy as jnp
import numpy as np
from jax import lax
from jax.experimental import pallas as pl
from jax.experimental.pallas import tpu as pltpu
from jax.experimental.pallas import tpu_sc as plsc

_KEY0 = 0
_KEY1 = 42
_KEY2 = _KEY0 ^ _KEY1 ^ 0x1BD11BDA

_TINY = np.float32(np.finfo(np.float32).tiny)
_ROT_A = (13, 15, 26, 6)
_ROT_B = (17, 29, 16, 24)

_TR = 8
_TC = 1024

_WS = 6912
_NSC = 32
_SC_COLS = _WS * _NSC
_BK = 8192
_SC_BLOCKS = _SC_COLS // _BK


def _i32(v):
    v = int(v) & 0xFFFFFFFF
    return jnp.int32(v - (1 << 32) if v >= (1 << 31) else v)


def _rotl(x, d):
    return (x << d) | lax.shift_right_logical(x, 32 - d)


def _threefry2x32_xored(x1):
    ks = (_KEY0, _KEY1, _KEY2)
    rots = (_ROT_A, _ROT_B)
    x0 = x1
    x1 = _rotl(x1, rots[0][0]) ^ x0
    for r in rots[0][1:]:
        x0 = x0 + x1
        x1 = _rotl(x1, r) ^ x0
    x0 = x0 + _i32(ks[1])
    x1 = x1 + _i32(ks[2] + 1)
    for i in range(1, 5):
        for r in rots[i % 2]:
            x0 = x0 + x1
            x1 = _rotl(x1, r) ^ x0
        x0 = x0 + _i32(ks[(i + 1) % 3])
        x1 = x1 + _i32(ks[(i + 2) % 3] + i + 1)
    return x0 ^ x1


def _bits_to_u(bits):
    fbits = lax.shift_right_logical(bits, 9) | jnp.int32(0x3F800000)
    f = lax.bitcast_convert_type(fbits, jnp.float32) - jnp.float32(1.0)
    return jnp.maximum(_TINY, f + _TINY)


def _tc_body(x_ref, oi_ref, ov_ref, vmax_ref, vidx_ref, *, nblocks,
             row_stride, valid_cols, bk, nfull, nskip):
    j = pl.program_id(0)

    @pl.when(j == 0)
    def _init():
        vmax_ref[...] = jnp.full_like(vmax_ref[...], -jnp.inf)
        vidx_ref[...] = jnp.zeros_like(vidx_ref[...])

    rows = x_ref.shape[0]
    pat = (lax.broadcasted_iota(jnp.int32, (_TR, _TC), 0)
           * jnp.int32(row_stride)
           + lax.broadcasted_iota(jnp.int32, (_TR, _TC), 1))
    cloc = lax.broadcasted_iota(jnp.int32, (_TR, _TC), 1)
    col0 = jnp.where(j < nfull, j, j + nskip) * bk
    big = jnp.int32(np.iinfo(np.int32).max)

    def tiles(masked):
        for rt in range(rows // _TR):
            for s in range(bk // _TC):
                x = x_ref[rt * _TR:(rt + 1) * _TR, s * _TC:(s + 1) * _TC]
                base = rt * _TR * row_stride + _KEY1
                x1 = pat + (col0 + s * _TC + base)
                u = _bits_to_u(_threefry2x32_xored(x1))
                score = x - jnp.log(-jnp.log(u))
                if masked:
                    score = jnp.where(
                        cloc < valid_cols - (col0 + s * _TC), score, -jnp.inf)
                m = jnp.max(score, axis=1, keepdims=True)
                cand = jnp.where(score == m, cloc, big)
                idx = jnp.min(cand, axis=1, keepdims=True) + (col0 + s * _TC)
                rs = slice(rt * _TR, (rt + 1) * _TR)
                better = m > vmax_ref[rs, :]
                vidx_ref[rs, :] = jnp.where(better, idx, vidx_ref[rs, :])
                vmax_ref[rs, :] = jnp.where(better, m, vmax_ref[rs, :])

    @pl.when(j < nblocks - 1)
    def _main():
        tiles(masked=False)

    @pl.when(j == nblocks - 1)
    def _tail():
        tiles(masked=True)
        oi_ref[...] = vidx_ref[...]
        ov_ref[...] = vmax_ref[...]


def _tc_sample(logits, nfull, nskip):
    rows, ncols = logits.shape
    bk = _BK
    nblocks = pl.cdiv(ncols, bk) - nskip
    idx, val = pl.pallas_call(
        functools.partial(_tc_body, nblocks=nblocks, row_stride=ncols,
                          valid_cols=ncols, bk=bk, nfull=nfull, nskip=nskip),
        grid=(nblocks,),
        in_specs=[pl.BlockSpec(
            (rows, bk),
            lambda j: (0, jnp.where(j < nfull, j, j + nskip)))],
        out_specs=[pl.BlockSpec((rows, 1), lambda j: (0, 0)),
                   pl.BlockSpec((rows, 1), lambda j: (0, 0))],
        out_shape=[jax.ShapeDtypeStruct((rows, 1), jnp.int32),
                   jax.ShapeDtypeStruct((rows, 1), jnp.float32)],
        scratch_shapes=[
            pltpu.VMEM((rows, 1), jnp.float32),
            pltpu.VMEM((rows, 1), jnp.int32),
        ],
    )(logits)
    return idx, val


_LOG_P = (7.0376836292e-2, -1.1514610310e-1, 1.1676998740e-1,
          -1.2420140846e-1, 1.4249322787e-1, -1.6668057665e-1,
          2.0000714765e-1, -2.4999993993e-1, 3.3333331174e-1)


def _logf(x):
    xb = lax.bitcast_convert_type(x, jnp.int32)
    e = lax.shift_right_logical(xb, 23) - jnp.int32(126)
    m = lax.bitcast_convert_type(
        (xb & jnp.int32(0x007FFFFF)) | jnp.int32(0x3F000000), jnp.float32)
    c = m < jnp.float32(0.70710678118654752440)
    xr = jnp.where(c, m + m, m) - jnp.float32(1.0)
    ef = (e - jnp.where(c, jnp.int32(1), jnp.int32(0))).astype(jnp.float32)
    y = jnp.float32(_LOG_P[0])
    for p in _LOG_P[1:]:
        y = y * xr + jnp.float32(p)
    z = xr * xr
    y = y * xr * z
    y = y + ef * jnp.float32(-2.12194440e-4)
    y = y - jnp.float32(0.5) * z
    return (xr + y) + ef * jnp.float32(0.693359375)


def _sc_row(buf_ref, vout_ref, iout_ref, rr, r, sbase, ncols):
    lanes = lax.iota(jnp.int32, 16)
    rowp = r * jnp.int32(ncols) + sbase + jnp.int32(_KEY1)

    def body(i, carry):
        at, ac, ax, ae = carry
        for t in range(4):
            off = i * 64 + t * 16
            x = buf_ref[rr, pl.ds(off, 16)]
            x1 = lanes + (rowp + off)
            u = _bits_to_u(_threefry2x32_xored(x1))
            e2 = jnp.float32(0.0) - _logf(u)
            tt = e2 * jnp.exp(jnp.float32(0.0) - x)
            cond = tt < at
            at = jnp.where(cond, tt, at)
            ac = jnp.where(cond, lanes + (sbase + off), ac)
            ax = jnp.where(cond, x, ax)
            ae = jnp.where(cond, e2, ae)
        return at, ac, ax, ae

    at0 = jnp.full((16,), jnp.inf, jnp.float32)
    ac0 = jnp.zeros((16,), jnp.int32)
    af0 = jnp.zeros((16,), jnp.float32)
    ae0 = jnp.full((16,), 1.0, jnp.float32)
    _, ac, ax, ae = lax.fori_loop(0, _WS // 64, body, (at0, ac0, af0, ae0))
    vout_ref[pl.ds(r * 16, 16)] = ax - _logf(ae)
    iout_ref[pl.ds(r * 16, 16)] = ac


def _sc_kernel_body(x_hbm, val_hbm, idx_hbm, bufa, bufb, vout, iout,
                    sema, semb, *, rows, ncols, col_base):
    wid = lax.axis_index("s") * 2 + lax.axis_index("c")
    sbase = jnp.int32(col_base) + wid * jnp.int32(_WS)
    ngrp = rows // 8

    def src(g):
        return x_hbm.at[pl.ds(g * 8, 8), pl.ds(sbase, _WS)]

    def do_group(buf_ref, g):
        def rowloop(rr, _):
            _sc_row(buf_ref, vout, iout, rr, g * 8 + rr, sbase, ncols)
            return 0
        lax.fori_loop(0, 8, rowloop, 0)

    pltpu.async_copy(src(0), bufa, sema)
    pltpu.async_copy(src(1), bufb, semb)

    def outer(g2, _):
        g0 = 2 * g2
        pltpu.make_async_copy(src(g0), bufa, sema).wait()
        do_group(bufa, g0)

        @pl.when(g2 < ngrp // 2 - 1)
        def _():
            pltpu.async_copy(src(g0 + 2), bufa, sema)

        pltpu.make_async_copy(src(g0 + 1), bufb, semb).wait()
        do_group(bufb, g0 + 1)

        @pl.when(g2 < ngrp // 2 - 1)
        def _():
            pltpu.async_copy(src(g0 + 3), bufb, semb)

        return 0

    lax.fori_loop(0, ngrp // 2, outer, 0)
    pltpu.sync_copy(vout, val_hbm.at[pl.ds(wid * (rows * 16), rows * 16)])
    pltpu.sync_copy(iout, idx_hbm.at[pl.ds(wid * (rows * 16), rows * 16)])


def _sc_sample(logits, col_base):
    rows, ncols = logits.shape
    mesh = plsc.VectorSubcoreMesh(core_axis_name="c", subcore_axis_name="s")
    k = pl.kernel(
        functools.partial(_sc_kernel_body, rows=rows, ncols=ncols,
                          col_base=col_base),
        out_type=[jax.ShapeDtypeStruct((_NSC * rows * 16,), jnp.float32),
                  jax.ShapeDtypeStruct((_NSC * rows * 16,), jnp.int32)],
        mesh=mesh,
        scratch_types=[
            pltpu.VMEM((8, _WS), jnp.float32),
            pltpu.VMEM((8, _WS), jnp.float32),
            pltpu.VMEM((rows * 16,), jnp.float32),
            pltpu.VMEM((rows * 16,), jnp.int32),
            pltpu.SemaphoreType.DMA,
            pltpu.SemaphoreType.DMA,
        ],
    )
    return k(logits)


@jax.jit
def kernel(logits):
    rows, ncols = logits.shape
    nfull = ncols // _BK - _SC_BLOCKS
    col_base = nfull * _BK
    tc_idx, tc_val = _tc_sample(logits, nfull, _SC_BLOCKS)
    sc_val, sc_idx = _sc_sample(logits, col_base)

    scv = sc_val.reshape(_NSC, rows, 16).transpose(1, 0, 2).reshape(rows, -1)
    sci = sc_idx.reshape(_NSC, rows, 16).transpose(1, 0, 2).reshape(rows, -1)
    cand_v = jnp.concatenate([tc_val, scv], axis=1)
    cand_i = jnp.concatenate([tc_idx, sci], axis=1)
    m = jnp.max(cand_v, axis=1, keepdims=True)
    big = jnp.int32(np.iinfo(np.int32).max)
    return jnp.min(jnp.where(cand_v == m, cand_i, big), axis=1)


# speedup vs baseline: 1.0124x; 1.0124x over previous
import functools

import jax
import jax.numpy as jnp
import numpy as np
from jax import lax
from jax.experimental import pallas as pl
from jax.experimental.pallas import tpu as pltpu
from jax.experimental.pallas import tpu_sc as plsc

_KEY0 = 0
_KEY1 = 42
_KEY2 = _KEY0 ^ _KEY1 ^ 0x1BD11BDA

_TINY = np.float32(np.finfo(np.float32).tiny)
_ROT_A = (13, 15, 26, 6)
_ROT_B = (17, 29, 16, 24)

_TR = 8
_TC = 1024

_WS = 6656
_NSC = 32
_SC_COLS = _WS * _NSC
_BK = 4096
_SC_BLOCKS = _SC_COLS // _BK


def _i32(v):
    v = int(v) & 0xFFFFFFFF
    return jnp.int32(v - (1 << 32) if v >= (1 << 31) else v)


def _rotl(x, d):
    return (x << d) | lax.shift_right_logical(x, 32 - d)


def _threefry2x32_xored(x1):
    ks = (_KEY0, _KEY1, _KEY2)
    rots = (_ROT_A, _ROT_B)
    x0 = x1
    x1 = _rotl(x1, rots[0][0]) ^ x0
    for r in rots[0][1:]:
        x0 = x0 + x1
        x1 = _rotl(x1, r) ^ x0
    x0 = x0 + _i32(ks[1])
    x1 = x1 + _i32(ks[2] + 1)
    for i in range(1, 5):
        for r in rots[i % 2]:
            x0 = x0 + x1
            x1 = _rotl(x1, r) ^ x0
        x0 = x0 + _i32(ks[(i + 1) % 3])
        x1 = x1 + _i32(ks[(i + 2) % 3] + i + 1)
    return x0 ^ x1


def _bits_to_u(bits):
    fbits = lax.shift_right_logical(bits, 9) | jnp.int32(0x3F800000)
    f = lax.bitcast_convert_type(fbits, jnp.float32) - jnp.float32(1.0)
    return jnp.maximum(_TINY, f + _TINY)


def _tc_body(x_ref, oi_ref, ov_ref, vmax_ref, vidx_ref, *, nblocks,
             row_stride, valid_cols, bk, nfull, nskip):
    j = pl.program_id(0)

    @pl.when(j == 0)
    def _init():
        vmax_ref[...] = jnp.full_like(vmax_ref[...], -jnp.inf)
        vidx_ref[...] = jnp.zeros_like(vidx_ref[...])

    rows = x_ref.shape[0]
    pat = (lax.broadcasted_iota(jnp.int32, (_TR, _TC), 0)
           * jnp.int32(row_stride)
           + lax.broadcasted_iota(jnp.int32, (_TR, _TC), 1))
    cloc = lax.broadcasted_iota(jnp.int32, (_TR, _TC), 1)
    col0 = jnp.where(j < nfull, j, j + nskip) * bk
    big = jnp.int32(np.iinfo(np.int32).max)

    def tiles(masked):
        for rt in range(rows // _TR):
            for s in range(bk // _TC):
                x = x_ref[rt * _TR:(rt + 1) * _TR, s * _TC:(s + 1) * _TC]
                base = rt * _TR * row_stride + _KEY1
                x1 = pat + (col0 + s * _TC + base)
                u = _bits_to_u(_threefry2x32_xored(x1))
                score = x - jnp.log(-jnp.log(u))
                if masked:
                    score = jnp.where(
                        cloc < valid_cols - (col0 + s * _TC), score, -jnp.inf)
                m = jnp.max(score, axis=1, keepdims=True)
                cand = jnp.where(score == m, cloc, big)
                idx = jnp.min(cand, axis=1, keepdims=True) + (col0 + s * _TC)
                rs = slice(rt * _TR, (rt + 1) * _TR)
                better = m > vmax_ref[rs, :]
                vidx_ref[rs, :] = jnp.where(better, idx, vidx_ref[rs, :])
                vmax_ref[rs, :] = jnp.where(better, m, vmax_ref[rs, :])

    @pl.when(j < nblocks - 1)
    def _main():
        tiles(masked=False)

    @pl.when(j == nblocks - 1)
    def _tail():
        tiles(masked=True)
        oi_ref[...] = vidx_ref[...]
        ov_ref[...] = vmax_ref[...]


def _tc_sample(logits, nfull, nskip):
    rows, ncols = logits.shape
    bk = _BK
    nblocks = pl.cdiv(ncols, bk) - nskip
    idx, val = pl.pallas_call(
        functools.partial(_tc_body, nblocks=nblocks, row_stride=ncols,
                          valid_cols=ncols, bk=bk, nfull=nfull, nskip=nskip),
        grid=(nblocks,),
        in_specs=[pl.BlockSpec(
            (rows, bk),
            lambda j: (0, jnp.where(j < nfull, j, j + nskip)))],
        out_specs=[pl.BlockSpec((rows, 1), lambda j: (0, 0)),
                   pl.BlockSpec((rows, 1), lambda j: (0, 0))],
        out_shape=[jax.ShapeDtypeStruct((rows, 1), jnp.int32),
                   jax.ShapeDtypeStruct((rows, 1), jnp.float32)],
        scratch_shapes=[
            pltpu.VMEM((rows, 1), jnp.float32),
            pltpu.VMEM((rows, 1), jnp.int32),
        ],
    )(logits)
    return idx, val


_LOG_P = (7.0376836292e-2, -1.1514610310e-1, 1.1676998740e-1,
          -1.2420140846e-1, 1.4249322787e-1, -1.6668057665e-1,
          2.0000714765e-1, -2.4999993993e-1, 3.3333331174e-1)


def _logf(x):
    xb = lax.bitcast_convert_type(x, jnp.int32)
    e = lax.shift_right_logical(xb, 23) - jnp.int32(126)
    m = lax.bitcast_convert_type(
        (xb & jnp.int32(0x007FFFFF)) | jnp.int32(0x3F000000), jnp.float32)
    c = m < jnp.float32(0.70710678118654752440)
    xr = jnp.where(c, m + m, m) - jnp.float32(1.0)
    ef = (e - jnp.where(c, jnp.int32(1), jnp.int32(0))).astype(jnp.float32)
    y = jnp.float32(_LOG_P[0])
    for p in _LOG_P[1:]:
        y = y * xr + jnp.float32(p)
    z = xr * xr
    y = y * xr * z
    y = y + ef * jnp.float32(-2.12194440e-4)
    y = y - jnp.float32(0.5) * z
    return (xr + y) + ef * jnp.float32(0.693359375)


def _sc_row(buf_ref, vout_ref, iout_ref, rr, r, sbase, ncols):
    lanes = lax.iota(jnp.int32, 16)
    rowp = r * jnp.int32(ncols) + sbase + jnp.int32(_KEY1)

    def body(i, carry):
        at, ac, ax, ae = carry
        for t in range(4):
            off = i * 64 + t * 16
            x = buf_ref[rr, pl.ds(off, 16)]
            x1 = lanes + (rowp + off)
            u = _bits_to_u(_threefry2x32_xored(x1))
            e2 = jnp.float32(0.0) - _logf(u)
            tt = e2 * jnp.exp(jnp.float32(0.0) - x)
            cond = tt < at
            at = jnp.where(cond, tt, at)
            ac = jnp.where(cond, lanes + (sbase + off), ac)
            ax = jnp.where(cond, x, ax)
            ae = jnp.where(cond, e2, ae)
        return at, ac, ax, ae

    at0 = jnp.full((16,), jnp.inf, jnp.float32)
    ac0 = jnp.zeros((16,), jnp.int32)
    af0 = jnp.zeros((16,), jnp.float32)
    ae0 = jnp.full((16,), 1.0, jnp.float32)
    _, ac, ax, ae = lax.fori_loop(0, _WS // 64, body, (at0, ac0, af0, ae0))
    vout_ref[pl.ds(r * 16, 16)] = ax - _logf(ae)
    iout_ref[pl.ds(r * 16, 16)] = ac


def _sc_kernel_body(x_hbm, val_hbm, idx_hbm, bufa, bufb, vout, iout,
                    sema, semb, *, rows, ncols, col_base):
    wid = lax.axis_index("s") * 2 + lax.axis_index("c")
    sbase = jnp.int32(col_base) + wid * jnp.int32(_WS)
    ngrp = rows // 8

    def src(g):
        return x_hbm.at[pl.ds(g * 8, 8), pl.ds(sbase, _WS)]

    def do_group(buf_ref, g):
        def rowloop(rr, _):
            _sc_row(buf_ref, vout, iout, rr, g * 8 + rr, sbase, ncols)
            return 0
        lax.fori_loop(0, 8, rowloop, 0)

    pltpu.async_copy(src(0), bufa, sema)
    pltpu.async_copy(src(1), bufb, semb)

    def outer(g2, _):
        g0 = 2 * g2
        pltpu.make_async_copy(src(g0), bufa, sema).wait()
        do_group(bufa, g0)

        @pl.when(g2 < ngrp // 2 - 1)
        def _():
            pltpu.async_copy(src(g0 + 2), bufa, sema)

        pltpu.make_async_copy(src(g0 + 1), bufb, semb).wait()
        do_group(bufb, g0 + 1)

        @pl.when(g2 < ngrp // 2 - 1)
        def _():
            pltpu.async_copy(src(g0 + 3), bufb, semb)

        return 0

    lax.fori_loop(0, ngrp // 2, outer, 0)
    pltpu.sync_copy(vout, val_hbm.at[pl.ds(wid * (rows * 16), rows * 16)])
    pltpu.sync_copy(iout, idx_hbm.at[pl.ds(wid * (rows * 16), rows * 16)])


def _sc_sample(logits, col_base):
    rows, ncols = logits.shape
    mesh = plsc.VectorSubcoreMesh(core_axis_name="c", subcore_axis_name="s")
    k = pl.kernel(
        functools.partial(_sc_kernel_body, rows=rows, ncols=ncols,
                          col_base=col_base),
        out_type=[jax.ShapeDtypeStruct((_NSC * rows * 16,), jnp.float32),
                  jax.ShapeDtypeStruct((_NSC * rows * 16,), jnp.int32)],
        mesh=mesh,
        scratch_types=[
            pltpu.VMEM((8, _WS), jnp.float32),
            pltpu.VMEM((8, _WS), jnp.float32),
            pltpu.VMEM((rows * 16,), jnp.float32),
            pltpu.VMEM((rows * 16,), jnp.int32),
            pltpu.SemaphoreType.DMA,
            pltpu.SemaphoreType.DMA,
        ],
    )
    return k(logits)


@jax.jit
def kernel(logits):
    rows, ncols = logits.shape
    nfull = ncols // _BK - _SC_BLOCKS
    col_base = nfull * _BK
    tc_idx, tc_val = _tc_sample(logits, nfull, _SC_BLOCKS)
    sc_val, sc_idx = _sc_sample(logits, col_base)

    scv = sc_val.reshape(_NSC, rows, 16).transpose(1, 0, 2).reshape(rows, -1)
    sci = sc_idx.reshape(_NSC, rows, 16).transpose(1, 0, 2).reshape(rows, -1)
    cand_v = jnp.concatenate([tc_val, scv], axis=1)
    cand_i = jnp.concatenate([tc_idx, sci], axis=1)
    m = jnp.max(cand_v, axis=1, keepdims=True)
    big = jnp.int32(np.iinfo(np.int32).max)
    return jnp.min(jnp.where(cand_v == m, cand_i, big), axis=1)


# speedup vs baseline: 1.0272x; 1.0146x over previous
import functools

import jax
import jax.numpy as jnp
import numpy as np
from jax import lax
from jax.experimental import pallas as pl
from jax.experimental.pallas import tpu as pltpu
from jax.experimental.pallas import tpu_sc as plsc

_KEY0 = 0
_KEY1 = 42
_KEY2 = _KEY0 ^ _KEY1 ^ 0x1BD11BDA

_TINY = np.float32(np.finfo(np.float32).tiny)
_ROT_A = (13, 15, 26, 6)
_ROT_B = (17, 29, 16, 24)

_TR = 8
_TC = 1024

_WS = 6656
_NSC = 32
_SC_COLS = _WS * _NSC
_BK = 8192
_SC_BLOCKS = _SC_COLS // _BK


def _i32(v):
    v = int(v) & 0xFFFFFFFF
    return jnp.int32(v - (1 << 32) if v >= (1 << 31) else v)


def _rotl(x, d):
    return (x << d) | lax.shift_right_logical(x, 32 - d)


def _threefry2x32_xored(x1):
    ks = (_KEY0, _KEY1, _KEY2)
    rots = (_ROT_A, _ROT_B)
    x0 = x1
    x1 = _rotl(x1, rots[0][0]) ^ x0
    for r in rots[0][1:]:
        x0 = x0 + x1
        x1 = _rotl(x1, r) ^ x0
    x0 = x0 + _i32(ks[1])
    x1 = x1 + _i32(ks[2] + 1)
    for i in range(1, 5):
        for r in rots[i % 2]:
            x0 = x0 + x1
            x1 = _rotl(x1, r) ^ x0
        x0 = x0 + _i32(ks[(i + 1) % 3])
        x1 = x1 + _i32(ks[(i + 2) % 3] + i + 1)
    return x0 ^ x1


def _bits_to_u(bits):
    fbits = lax.shift_right_logical(bits, 9) | jnp.int32(0x3F800000)
    f = lax.bitcast_convert_type(fbits, jnp.float32) - jnp.float32(1.0)
    return jnp.maximum(_TINY, f + _TINY)


def _tc_body(x_ref, oi_ref, ov_ref, vmax_ref, vidx_ref, *, nblocks,
             row_stride, valid_cols, bk, nfull, nskip):
    j = pl.program_id(0)

    @pl.when(j == 0)
    def _init():
        vmax_ref[...] = jnp.full_like(vmax_ref[...], -jnp.inf)
        vidx_ref[...] = jnp.zeros_like(vidx_ref[...])

    rows = x_ref.shape[0]
    pat = (lax.broadcasted_iota(jnp.int32, (_TR, _TC), 0)
           * jnp.int32(row_stride)
           + lax.broadcasted_iota(jnp.int32, (_TR, _TC), 1))
    cloc = lax.broadcasted_iota(jnp.int32, (_TR, _TC), 1)
    col0 = jnp.where(j < nfull, j, j + nskip) * bk
    big = jnp.int32(np.iinfo(np.int32).max)

    def tiles(masked):
        for rt in range(rows // _TR):
            for s in range(bk // _TC):
                x = x_ref[rt * _TR:(rt + 1) * _TR, s * _TC:(s + 1) * _TC]
                base = rt * _TR * row_stride + _KEY1
                x1 = pat + (col0 + s * _TC + base)
                u = _bits_to_u(_threefry2x32_xored(x1))
                score = x - jnp.log(-jnp.log(u))
                if masked:
                    score = jnp.where(
                        cloc < valid_cols - (col0 + s * _TC), score, -jnp.inf)
                m = jnp.max(score, axis=1, keepdims=True)
                cand = jnp.where(score == m, cloc, big)
                idx = jnp.min(cand, axis=1, keepdims=True) + (col0 + s * _TC)
                rs = slice(rt * _TR, (rt + 1) * _TR)
                better = m > vmax_ref[rs, :]
                vidx_ref[rs, :] = jnp.where(better, idx, vidx_ref[rs, :])
                vmax_ref[rs, :] = jnp.where(better, m, vmax_ref[rs, :])

    @pl.when(j < nblocks - 1)
    def _main():
        tiles(masked=False)

    @pl.when(j == nblocks - 1)
    def _tail():
        tiles(masked=True)
        oi_ref[...] = vidx_ref[...]
        ov_ref[...] = vmax_ref[...]


def _tc_sample(logits, nfull, nskip):
    rows, ncols = logits.shape
    bk = _BK
    nblocks = pl.cdiv(ncols, bk) - nskip
    idx, val = pl.pallas_call(
        functools.partial(_tc_body, nblocks=nblocks, row_stride=ncols,
                          valid_cols=ncols, bk=bk, nfull=nfull, nskip=nskip),
        grid=(nblocks,),
        in_specs=[pl.BlockSpec(
            (rows, bk),
            lambda j: (0, jnp.where(j < nfull, j, j + nskip)))],
        out_specs=[pl.BlockSpec((rows, 1), lambda j: (0, 0)),
                   pl.BlockSpec((rows, 1), lambda j: (0, 0))],
        out_shape=[jax.ShapeDtypeStruct((rows, 1), jnp.int32),
                   jax.ShapeDtypeStruct((rows, 1), jnp.float32)],
        scratch_shapes=[
            pltpu.VMEM((rows, 1), jnp.float32),
            pltpu.VMEM((rows, 1), jnp.int32),
        ],
    )(logits)
    return idx, val


_LOG_P = (7.0376836292e-2, -1.1514610310e-1, 1.1676998740e-1,
          -1.2420140846e-1, 1.4249322787e-1, -1.6668057665e-1,
          2.0000714765e-1, -2.4999993993e-1, 3.3333331174e-1)


def _logf(x):
    xb = lax.bitcast_convert_type(x, jnp.int32)
    e = lax.shift_right_logical(xb, 23) - jnp.int32(126)
    m = lax.bitcast_convert_type(
        (xb & jnp.int32(0x007FFFFF)) | jnp.int32(0x3F000000), jnp.float32)
    c = m < jnp.float32(0.70710678118654752440)
    xr = jnp.where(c, m + m, m) - jnp.float32(1.0)
    ef = (e - jnp.where(c, jnp.int32(1), jnp.int32(0))).astype(jnp.float32)
    y = jnp.float32(_LOG_P[0])
    for p in _LOG_P[1:]:
        y = y * xr + jnp.float32(p)
    z = xr * xr
    y = y * xr * z
    y = y + ef * jnp.float32(-2.12194440e-4)
    y = y - jnp.float32(0.5) * z
    return (xr + y) + ef * jnp.float32(0.693359375)


def _sc_row(buf_ref, vout_ref, iout_ref, rr, r, sbase, ncols):
    lanes = lax.iota(jnp.int32, 16)
    rowp = r * jnp.int32(ncols) + sbase + jnp.int32(_KEY1)

    def body(i, carry):
        at, ac, ax, ae = carry
        for t in range(4):
            off = i * 64 + t * 16
            x = buf_ref[rr, pl.ds(off, 16)]
            x1 = lanes + (rowp + off)
            u = _bits_to_u(_threefry2x32_xored(x1))
            e2 = jnp.float32(0.0) - _logf(u)
            tt = e2 * jnp.exp(jnp.float32(0.0) - x)
            cond = tt < at
            at = jnp.where(cond, tt, at)
            ac = jnp.where(cond, lanes + (sbase + off), ac)
            ax = jnp.where(cond, x, ax)
            ae = jnp.where(cond, e2, ae)
        return at, ac, ax, ae

    at0 = jnp.full((16,), jnp.inf, jnp.float32)
    ac0 = jnp.zeros((16,), jnp.int32)
    af0 = jnp.zeros((16,), jnp.float32)
    ae0 = jnp.full((16,), 1.0, jnp.float32)
    _, ac, ax, ae = lax.fori_loop(0, _WS // 64, body, (at0, ac0, af0, ae0))
    vout_ref[pl.ds(r * 16, 16)] = ax - _logf(ae)
    iout_ref[pl.ds(r * 16, 16)] = ac


def _sc_kernel_body(x_hbm, val_hbm, idx_hbm, bufa, bufb, vout, iout,
                    sema, semb, *, rows, ncols, col_base):
    wid = lax.axis_index("s") * 2 + lax.axis_index("c")
    sbase = jnp.int32(col_base) + wid * jnp.int32(_WS)
    ngrp = rows // 8

    def src(g):
        return x_hbm.at[pl.ds(g * 8, 8), pl.ds(sbase, _WS)]

    def do_group(buf_ref, g):
        def rowloop(rr, _):
            _sc_row(buf_ref, vout, iout, rr, g * 8 + rr, sbase, ncols)
            return 0
        lax.fori_loop(0, 8, rowloop, 0)

    pltpu.async_copy(src(0), bufa, sema)
    pltpu.async_copy(src(1), bufb, semb)

    def outer(g2, _):
        g0 = 2 * g2
        pltpu.make_async_copy(src(g0), bufa, sema).wait()
        do_group(bufa, g0)

        @pl.when(g2 < ngrp // 2 - 1)
        def _():
            pltpu.async_copy(src(g0 + 2), bufa, sema)

        pltpu.make_async_copy(src(g0 + 1), bufb, semb).wait()
        do_group(bufb, g0 + 1)

        @pl.when(g2 < ngrp // 2 - 1)
        def _():
            pltpu.async_copy(src(g0 + 3), bufb, semb)

        return 0

    lax.fori_loop(0, ngrp // 2, outer, 0)
    pltpu.sync_copy(vout, val_hbm.at[pl.ds(wid * (rows * 16), rows * 16)])
    pltpu.sync_copy(iout, idx_hbm.at[pl.ds(wid * (rows * 16), rows * 16)])


def _sc_sample(logits, col_base):
    rows, ncols = logits.shape
    mesh = plsc.VectorSubcoreMesh(core_axis_name="c", subcore_axis_name="s")
    k = pl.kernel(
        functools.partial(_sc_kernel_body, rows=rows, ncols=ncols,
                          col_base=col_base),
        out_type=[jax.ShapeDtypeStruct((_NSC * rows * 16,), jnp.float32),
                  jax.ShapeDtypeStruct((_NSC * rows * 16,), jnp.int32)],
        mesh=mesh,
        scratch_types=[
            pltpu.VMEM((8, _WS), jnp.float32),
            pltpu.VMEM((8, _WS), jnp.float32),
            pltpu.VMEM((rows * 16,), jnp.float32),
            pltpu.VMEM((rows * 16,), jnp.int32),
            pltpu.SemaphoreType.DMA,
            pltpu.SemaphoreType.DMA,
        ],
    )
    return k(logits)


@jax.jit
def kernel(logits):
    rows, ncols = logits.shape
    nfull = ncols // _BK - _SC_BLOCKS
    col_base = nfull * _BK
    tc_idx, tc_val = _tc_sample(logits, nfull, _SC_BLOCKS)
    sc_val, sc_idx = _sc_sample(logits, col_base)

    scv = sc_val.reshape(_NSC, rows, 16).transpose(1, 0, 2).reshape(rows, -1)
    sci = sc_idx.reshape(_NSC, rows, 16).transpose(1, 0, 2).reshape(rows, -1)
    cand_v = jnp.concatenate([tc_val, scv], axis=1)
    cand_i = jnp.concatenate([tc_idx, sci], axis=1)
    m = jnp.max(cand_v, axis=1, keepdims=True)
    big = jnp.int32(np.iinfo(np.int32).max)
    return jnp.min(jnp.where(cand_v == m, cand_i, big), axis=1)
